# baseline (speedup 1.0000x reference)
import functools

import jax
import jax.numpy as jnp
from jax import lax
from jax.experimental import pallas as pl
from jax.experimental.pallas import tpu as pltpu
from jax.experimental.pallas import tpu_sc as plsc

NC = 2
NS = 16
NW = NC * NS


LANES = 256
SPILL = 8


def _deg_kernel_call(src_pad, zeros_n, ones_c, n, rpw):
    zstripe = n // 10
    mesh = plsc.VectorSubcoreMesh(core_axis_name="c", subcore_axis_name="s")

    @functools.partial(
        pl.kernel,
        out_type=jax.ShapeDtypeStruct((NC * n,), jnp.float32),
        mesh=mesh,
        scratch_types=[
            pltpu.VMEM((rpw, LANES), jnp.int32),
            pltpu.VMEM((LANES,), jnp.float32),
            pltpu.VMEM((zstripe,), jnp.float32),
            pltpu.VMEM_SHARED((n + SPILL,), jnp.float32),
        ],
        compiler_params=pltpu.CompilerParams(use_tc_tiling_on_sc=False),
    )
    def deg_kernel(src_hbm, zeros_hbm, ones_hbm, out_hbm, idx_v, ones_v,
                   stripe_v, acc):
        cid = lax.axis_index("c")
        sid = lax.axis_index("s")
        wid = sid * NC + cid

        pltpu.sync_copy(src_hbm.at[pl.ds(wid * rpw, rpw)], idx_v)
        pltpu.sync_copy(ones_hbm, ones_v)

        @pl.when(sid < 10)
        def _zero():
            pltpu.sync_copy(zeros_hbm.at[pl.ds(sid * zstripe, zstripe)],
                            stripe_v)
            pltpu.sync_copy(stripe_v, acc.at[pl.ds(sid * zstripe, zstripe)])

        plsc.subcore_barrier()

        def body(i, carry):
            pltpu.sync_copy(ones_v, acc.at[idx_v.at[i]], add=True)
            return carry

        lax.fori_loop(0, rpw, body, 0)
        plsc.subcore_barrier()

        @pl.when(sid < 10)
        def _write():
            pltpu.sync_copy(acc.at[pl.ds(sid * zstripe, zstripe)], stripe_v)
            pltpu.sync_copy(stripe_v,
                            out_hbm.at[pl.ds(cid * n + sid * zstripe, zstripe)])

    return deg_kernel(src_pad, zeros_n, ones_c)


def _segsum_kernel_call(table, pidx, zeros_nd, n, d, rpw):
    rps = n // NS
    mesh = plsc.VectorSubcoreMesh(core_axis_name="c", subcore_axis_name="s")

    @functools.partial(
        pl.kernel,
        out_type=jax.ShapeDtypeStruct((NC * n, d), jnp.float32),
        mesh=mesh,
        scratch_types=(
            [pltpu.VMEM((2, rpw, LANES), jnp.int32),
             pltpu.VMEM((rps, d), jnp.float32),
             pltpu.VMEM_SHARED((n + SPILL, d), jnp.float32)]
            + [pltpu.VMEM((LANES, d), jnp.float32)]
            + [pltpu.SemaphoreType.DMA]
        ),
        compiler_params=pltpu.CompilerParams(use_tc_tiling_on_sc=False),
    )
    def seg_kernel(table_hbm, pidx_hbm, zeros_hbm, out_hbm,
                   pidx_v, stripe_v, acc, *bufs):
        rows = bufs[0]
        sem_g = bufs[1]
        cid = lax.axis_index("c")
        sid = lax.axis_index("s")
        wid = sid * NC + cid

        pltpu.sync_copy(pidx_hbm.at[:, pl.ds(wid * rpw, rpw), :], pidx_v)

        pltpu.sync_copy(zeros_hbm.at[pl.ds(sid * rps, rps)], stripe_v)
        pltpu.sync_copy(stripe_v, acc.at[pl.ds(sid * rps, rps)])
        plsc.subcore_barrier()

        def body(g, carry):
            pltpu.async_copy(table_hbm.at[pidx_v.at[0, g]],
                             rows, sem_g).wait()
            pltpu.sync_copy(rows, acc.at[pidx_v.at[1, g]], add=True)
            return carry

        lax.fori_loop(0, rpw, body, 0)
        plsc.subcore_barrier()

        pltpu.sync_copy(acc.at[pl.ds(sid * rps, rps)], stripe_v)
        pltpu.sync_copy(stripe_v,
                        out_hbm.at[pl.ds(cid * n + sid * rps, rps)])

    return seg_kernel(table, pidx, zeros_nd)


def _dis_from_deg(deg_blk):
    deg = deg_blk[:, 0:1] + deg_blk[:, 1:2]
    safe = jnp.where(deg > 0, deg, 1.0)
    return jnp.where(deg > 0, lax.rsqrt(safe), 0.0)


def _tc1_call(x, wcat, deg_t, n, f, hid, r):
    def body(x_ref, w_ref, deg_ref, y1_ref, z1_ref):
        xw = jnp.dot(x_ref[...], w_ref[...],
                     preferred_element_type=jnp.float32)
        dis = _dis_from_deg(deg_ref[...])
        y1_ref[...] = xw[:, :hid]
        z1_ref[...] = xw[:, hid:] * dis

    return pl.pallas_call(
        body,
        grid=(n // r,),
        in_specs=[
            pl.BlockSpec((r, f), lambda i: (i, 0)),
            pl.BlockSpec((f, 2 * hid), lambda i: (0, 0)),
            pl.BlockSpec((r, 2), lambda i: (i, 0)),
        ],
        out_specs=[
            pl.BlockSpec((r, hid), lambda i: (i, 0)),
            pl.BlockSpec((r, hid), lambda i: (i, 0)),
        ],
        out_shape=[
            jax.ShapeDtypeStruct((n, hid), jnp.float32),
            jax.ShapeDtypeStruct((n, hid), jnp.float32),
        ],
    )(x, wcat, deg_t)


def _tc2_call(y1, s1p, deg_t, b1, wcat2, n, hid, ncls, dpad, r):
    wcols = ncls + dpad

    def body(y1_ref, s_ref, deg_ref, b_ref, w_ref, y2_ref, z2_ref):
        dis = _dis_from_deg(deg_ref[...])
        s = s_ref[0] + s_ref[1]
        h = jax.nn.relu(y1_ref[...] - dis * s + b_ref[...])
        hw = jnp.dot(h, w_ref[...], preferred_element_type=jnp.float32)
        y2_ref[...] = hw[:, :ncls]
        z2_ref[...] = hw[:, ncls:] * dis

    return pl.pallas_call(
        body,
        grid=(n // r,),
        in_specs=[
            pl.BlockSpec((r, hid), lambda i: (i, 0)),
            pl.BlockSpec((2, r, hid), lambda i: (0, i, 0)),
            pl.BlockSpec((r, 2), lambda i: (i, 0)),
            pl.BlockSpec((1, hid), lambda i: (0, 0)),
            pl.BlockSpec((hid, wcols), lambda i: (0, 0)),
        ],
        out_specs=[
            pl.BlockSpec((r, ncls), lambda i: (i, 0)),
            pl.BlockSpec((r, dpad), lambda i: (i, 0)),
        ],
        out_shape=[
            jax.ShapeDtypeStruct((n, ncls), jnp.float32),
            jax.ShapeDtypeStruct((n, dpad), jnp.float32),
        ],
    )(y1, s1p, deg_t, b1, wcat2)


def _tc3_call(y2, s2p, deg_t, b2, n, ncls, dpad, r):
    def body(y2_ref, s_ref, deg_ref, b_ref, out_ref):
        dis = _dis_from_deg(deg_ref[...])
        s = (s_ref[0] + s_ref[1])[:, :ncls]
        logits = y2_ref[...] - dis * s + b_ref[...]
        m = jnp.max(logits, axis=1, keepdims=True)
        shifted = logits - m
        lse = jnp.log(jnp.sum(jnp.exp(shifted), axis=1, keepdims=True))
        out_ref[...] = shifted - lse

    return pl.pallas_call(
        body,
        grid=(n // r,),
        in_specs=[
            pl.BlockSpec((r, ncls), lambda i: (i, 0)),
            pl.BlockSpec((2, r, dpad), lambda i: (0, i, 0)),
            pl.BlockSpec((r, 2), lambda i: (i, 0)),
            pl.BlockSpec((1, ncls), lambda i: (0, 0)),
        ],
        out_specs=pl.BlockSpec((r, ncls), lambda i: (i, 0)),
        out_shape=jax.ShapeDtypeStruct((n, ncls), jnp.float32),
    )(y2, s2p, deg_t, b2)


def kernel(x, edge_index, W0_1, W1_1, b1, W0_2, W1_2, b2):
    n, f = x.shape
    e = edge_index.shape[1]
    hid = W0_1.shape[1]
    ncls = W0_2.shape[1]
    dpad = 40
    r = 1000

    rows_tot = e // LANES
    rpw = -(-rows_tot // NW)
    pad_rows = NW * rpw - rows_tot
    src2d = edge_index[0].reshape(rows_tot, LANES)
    dst2d = edge_index[1].reshape(rows_tot, LANES)
    pad_spill = n + (jnp.arange(pad_rows * LANES, dtype=jnp.int32)
                     % SPILL).reshape(pad_rows, LANES)
    src_g = jnp.pad(src2d, ((0, pad_rows), (0, 0)))
    dst_s = jnp.concatenate([dst2d, pad_spill], axis=0)
    pidx = jnp.stack([src_g, dst_s])
    src_deg = jnp.concatenate([src2d, pad_spill], axis=0)

    zeros_n = jnp.zeros((n,), jnp.float32)
    ones_c = jnp.ones((LANES,), jnp.float32)
    deg_p = _deg_kernel_call(src_deg, zeros_n, ones_c, n, rpw)
    deg_t = deg_p.reshape(NC, n).T

    wcat1 = jnp.concatenate([W0_1, W1_1], axis=1)
    y1, z1 = _tc1_call(x, wcat1, deg_t, n, f, hid, r)

    zeros_nh = jnp.zeros((n, hid), jnp.float32)
    s1p = _segsum_kernel_call(z1, pidx, zeros_nh, n, hid, rpw)
    s1p = s1p.reshape(NC, n, hid)

    wcat2 = jnp.concatenate(
        [W0_2, W1_2, jnp.zeros((hid, dpad - ncls), jnp.float32)], axis=1)
    y2, z2 = _tc2_call(y1, s1p, deg_t, b1.reshape(1, hid), wcat2,
                       n, hid, ncls, dpad, r)

    zeros_nd = jnp.zeros((n, dpad), jnp.float32)
    s2p = _segsum_kernel_call(z2, pidx, zeros_nd, n, dpad, rpw)
    s2p = s2p.reshape(NC, n, dpad)

    return _tc3_call(y2, s2p, deg_t, b2.reshape(1, ncls), n, ncls, dpad, r)


# speedup vs baseline: 1.1896x; 1.1896x over previous
import functools

import jax
import jax.numpy as jnp
from jax import lax
from jax.experimental import pallas as pl
from jax.experimental.pallas import tpu as pltpu
from jax.experimental.pallas import tpu_sc as plsc

NC = 2
NS = 16
NW = NC * NS


LANES = 128
SPILL = 8


def _deg_kernel_call(src_pad, zeros_n, ones_c, n, rpw):
    zstripe = n // 10
    mesh = plsc.VectorSubcoreMesh(core_axis_name="c", subcore_axis_name="s")

    @functools.partial(
        pl.kernel,
        out_type=jax.ShapeDtypeStruct((NC * n,), jnp.float32),
        mesh=mesh,
        scratch_types=[
            pltpu.VMEM((rpw, LANES), jnp.int32),
            pltpu.VMEM((LANES,), jnp.float32),
            pltpu.VMEM((zstripe,), jnp.float32),
            pltpu.VMEM_SHARED((n + SPILL,), jnp.float32),
        ],
        compiler_params=pltpu.CompilerParams(use_tc_tiling_on_sc=False),
    )
    def deg_kernel(src_hbm, zeros_hbm, ones_hbm, out_hbm, idx_v, ones_v,
                   stripe_v, acc):
        cid = lax.axis_index("c")
        sid = lax.axis_index("s")
        wid = sid * NC + cid

        pltpu.sync_copy(src_hbm.at[pl.ds(wid * rpw, rpw)], idx_v)
        pltpu.sync_copy(ones_hbm, ones_v)

        @pl.when(sid < 10)
        def _zero():
            pltpu.sync_copy(zeros_hbm.at[pl.ds(sid * zstripe, zstripe)],
                            stripe_v)
            pltpu.sync_copy(stripe_v, acc.at[pl.ds(sid * zstripe, zstripe)])

        plsc.subcore_barrier()

        def body(i, carry):
            pltpu.sync_copy(ones_v, acc.at[idx_v.at[i]], add=True)
            return carry

        lax.fori_loop(0, rpw, body, 0)
        plsc.subcore_barrier()

        @pl.when(sid < 10)
        def _write():
            pltpu.sync_copy(acc.at[pl.ds(sid * zstripe, zstripe)], stripe_v)
            pltpu.sync_copy(stripe_v,
                            out_hbm.at[pl.ds(cid * n + sid * zstripe, zstripe)])

    return deg_kernel(src_pad, zeros_n, ones_c)


def _segsum_kernel_call(table, pidx, zeros_nd, n, d, rpw):
    U = 1
    rps = n // NS
    mesh = plsc.VectorSubcoreMesh(core_axis_name="c", subcore_axis_name="s")

    @functools.partial(
        pl.kernel,
        out_type=jax.ShapeDtypeStruct((NC * n, d), jnp.float32),
        mesh=mesh,
        scratch_types=(
            [pltpu.VMEM((2, rpw, LANES), jnp.int32),
             pltpu.VMEM((rps, d), jnp.float32),
             pltpu.VMEM_SHARED((n + SPILL, d), jnp.float32)]
            + [pltpu.VMEM((LANES, d), jnp.float32)]
            + [pltpu.SemaphoreType.DMA]
        ),
        compiler_params=pltpu.CompilerParams(use_tc_tiling_on_sc=False),
    )
    def seg_kernel(table_hbm, pidx_hbm, zeros_hbm, out_hbm,
                   pidx_v, stripe_v, acc, *bufs):
        rows = bufs[0]
        sem_g = bufs[1]
        cid = lax.axis_index("c")
        sid = lax.axis_index("s")
        wid = sid * NC + cid

        pltpu.sync_copy(pidx_hbm.at[:, pl.ds(wid * rpw, rpw), :], pidx_v)

        pltpu.sync_copy(zeros_hbm.at[pl.ds(sid * rps, rps)], stripe_v)
        pltpu.sync_copy(stripe_v, acc.at[pl.ds(sid * rps, rps)])
        plsc.subcore_barrier()

        def body(g, carry):
            pltpu.async_copy(table_hbm.at[pidx_v.at[0, g]],
                             rows, sem_g).wait()
            pltpu.sync_copy(rows, acc.at[pidx_v.at[1, g]], add=True)
            return carry

        lax.fori_loop(0, rpw, body, 0)
        plsc.subcore_barrier()

        pltpu.sync_copy(acc.at[pl.ds(sid * rps, rps)], stripe_v)
        pltpu.sync_copy(stripe_v,
                        out_hbm.at[pl.ds(cid * n + sid * rps, rps)])

    return seg_kernel(table, pidx, zeros_nd)


def _dis_from_deg(deg_blk):
    deg = deg_blk[:, 0:1] + deg_blk[:, 1:2]
    safe = jnp.where(deg > 0, deg, 1.0)
    return jnp.where(deg > 0, lax.rsqrt(safe), 0.0)


def _tc1_call(x, wcat, deg_t, n, f, hid, r):
    def body(x_ref, w_ref, deg_ref, y1_ref, z1_ref):
        xw = jnp.dot(x_ref[...], w_ref[...],
                     preferred_element_type=jnp.float32)
        dis = _dis_from_deg(deg_ref[...])
        y1_ref[...] = xw[:, :hid]
        z1_ref[...] = xw[:, hid:] * dis

    return pl.pallas_call(
        body,
        grid=(n // r,),
        in_specs=[
            pl.BlockSpec((r, f), lambda i: (i, 0)),
            pl.BlockSpec((f, 2 * hid), lambda i: (0, 0)),
            pl.BlockSpec((r, 2), lambda i: (i, 0)),
        ],
        out_specs=[
            pl.BlockSpec((r, hid), lambda i: (i, 0)),
            pl.BlockSpec((r, hid), lambda i: (i, 0)),
        ],
        out_shape=[
            jax.ShapeDtypeStruct((n, hid), jnp.float32),
            jax.ShapeDtypeStruct((n, hid), jnp.float32),
        ],
    )(x, wcat, deg_t)


def _tc2_call(y1, s1p, deg_t, b1, wcat2, n, hid, ncls, dpad, r):
    wcols = ncls + dpad

    def body(y1_ref, s_ref, deg_ref, b_ref, w_ref, y2_ref, z2_ref):
        dis = _dis_from_deg(deg_ref[...])
        s = s_ref[0] + s_ref[1]
        h = jax.nn.relu(y1_ref[...] - dis * s + b_ref[...])
        hw = jnp.dot(h, w_ref[...], preferred_element_type=jnp.float32)
        y2_ref[...] = hw[:, :ncls]
        z2_ref[...] = hw[:, ncls:] * dis

    return pl.pallas_call(
        body,
        grid=(n // r,),
        in_specs=[
            pl.BlockSpec((r, hid), lambda i: (i, 0)),
            pl.BlockSpec((2, r, hid), lambda i: (0, i, 0)),
            pl.BlockSpec((r, 2), lambda i: (i, 0)),
            pl.BlockSpec((1, hid), lambda i: (0, 0)),
            pl.BlockSpec((hid, wcols), lambda i: (0, 0)),
        ],
        out_specs=[
            pl.BlockSpec((r, ncls), lambda i: (i, 0)),
            pl.BlockSpec((r, dpad), lambda i: (i, 0)),
        ],
        out_shape=[
            jax.ShapeDtypeStruct((n, ncls), jnp.float32),
            jax.ShapeDtypeStruct((n, dpad), jnp.float32),
        ],
    )(y1, s1p, deg_t, b1, wcat2)


def _tc3_call(y2, s2p, deg_t, b2, n, ncls, dpad, r):
    def body(y2_ref, s_ref, deg_ref, b_ref, out_ref):
        dis = _dis_from_deg(deg_ref[...])
        s = (s_ref[0] + s_ref[1])[:, :ncls]
        logits = y2_ref[...] - dis * s + b_ref[...]
        m = jnp.max(logits, axis=1, keepdims=True)
        shifted = logits - m
        lse = jnp.log(jnp.sum(jnp.exp(shifted), axis=1, keepdims=True))
        out_ref[...] = shifted - lse

    return pl.pallas_call(
        body,
        grid=(n // r,),
        in_specs=[
            pl.BlockSpec((r, ncls), lambda i: (i, 0)),
            pl.BlockSpec((2, r, dpad), lambda i: (0, i, 0)),
            pl.BlockSpec((r, 2), lambda i: (i, 0)),
            pl.BlockSpec((1, ncls), lambda i: (0, 0)),
        ],
        out_specs=pl.BlockSpec((r, ncls), lambda i: (i, 0)),
        out_shape=jax.ShapeDtypeStruct((n, ncls), jnp.float32),
    )(y2, s2p, deg_t, b2)


def kernel(x, edge_index, W0_1, W1_1, b1, W0_2, W1_2, b2):
    n, f = x.shape
    e = edge_index.shape[1]
    hid = W0_1.shape[1]
    ncls = W0_2.shape[1]
    dpad = 40
    r = 1000

    rows_tot = e // LANES
    rpw = -(-rows_tot // NW)
    pad_rows = NW * rpw - rows_tot
    src2d = edge_index[0].reshape(rows_tot, LANES)
    dst2d = edge_index[1].reshape(rows_tot, LANES)
    pad_spill = n + (jnp.arange(pad_rows * LANES, dtype=jnp.int32)
                     % SPILL).reshape(pad_rows, LANES)
    src_g = jnp.pad(src2d, ((0, pad_rows), (0, 0)))
    dst_s = jnp.concatenate([dst2d, pad_spill], axis=0)
    pidx = jnp.stack([src_g, dst_s])
    src_deg = jnp.concatenate([src2d, pad_spill], axis=0)

    zeros_n = jnp.zeros((n,), jnp.float32)
    ones_c = jnp.ones((LANES,), jnp.float32)
    deg_p = _deg_kernel_call(src_deg, zeros_n, ones_c, n, rpw)
    deg_t = deg_p.reshape(NC, n).T

    wcat1 = jnp.concatenate([W0_1, W1_1], axis=1)
    y1, z1 = _tc1_call(x, wcat1, deg_t, n, f, hid, r)

    zeros_nh = jnp.zeros((n, hid), jnp.float32)
    s1p = _segsum_kernel_call(z1, pidx, zeros_nh, n, hid, rpw)
    s1p = s1p.reshape(NC, n, hid)

    wcat2 = jnp.concatenate(
        [W0_2, W1_2, jnp.zeros((hid, dpad - ncls), jnp.float32)], axis=1)
    y2, z2 = _tc2_call(y1, s1p, deg_t, b1.reshape(1, hid), wcat2,
                       n, hid, ncls, dpad, r)

    zeros_nd = jnp.zeros((n, dpad), jnp.float32)
    s2p = _segsum_kernel_call(z2, pidx, zeros_nd, n, dpad, rpw)
    s2p = s2p.reshape(NC, n, dpad)

    return _tc3_call(y2, s2p, deg_t, b2.reshape(1, ncls), n, ncls, dpad, r)
